# baseline (device time: 44668 ns/iter reference)
import jax
import jax.numpy as jnp
from jax import lax
from jax.experimental import pallas as pl
from jax.experimental.pallas import tpu as pltpu

N_DEV = 32
N_TOK = 1024
D_MODEL = 256
D_FF = 512
E_LOC = 4
N_EXP = 128
BLK = N_TOK // N_DEV
MAX_M = 160


def kernel(x, router_W, route_idx, expert_W, shared_W):
    my = lax.axis_index("i")

    idx0 = route_idx[:, 0]
    owner = idx0 // E_LOC
    mine = owner == my
    tok = jnp.flatnonzero(mine, size=MAX_M, fill_value=N_TOK)
    m_count = jnp.sum(mine).astype(jnp.int32)
    tok_safe = jnp.minimum(tok, N_TOK - 1)
    xg = x[tok_safe]
    idxg = idx0[tok_safe][:, None]
    dstv = tok_safe // BLK
    valid = jnp.arange(MAX_M) < m_count
    n_send = jnp.sum(valid & (dstv != my)).astype(jnp.int32)
    blk_owner = lax.dynamic_slice(owner, (my * BLK,), (BLK,))
    n_remote = (BLK - jnp.sum(blk_owner == my)).astype(jnp.int32)
    meta = jnp.stack([m_count, n_send, n_remote]).astype(jnp.int32)
    x_mine = lax.dynamic_slice(x, (my * BLK, 0), (BLK, D_MODEL))

    def body(tok_sm, meta_sm, xg_ref, idxg_ref, xm_ref, rw_ref, ew_ref,
             sw_ref, out_ref, partial_ref, comm_ref, ssem, rsem):
        me = lax.axis_index("i")
        m_cnt = meta_sm[0]
        n_snd = meta_sm[1]
        n_rcv = meta_sm[2]

        barrier_sem = pltpu.get_barrier_semaphore()
        for k in range(1, N_DEV):
            nbr = lax.rem(me + k, N_DEV)
            pl.semaphore_signal(barrier_sem, inc=1, device_id=(nbr,),
                                device_id_type=pl.DeviceIdType.MESH)
        pl.semaphore_wait(barrier_sem, N_DEV - 1)

        xgb = xg_ref[:, :].astype(jnp.bfloat16)
        scores = jnp.dot(xgb, rw_ref[:, :].astype(jnp.bfloat16),
                         preferred_element_type=jnp.float32)
        s_max = jnp.max(scores, axis=-1, keepdims=True)
        p = jnp.exp(scores - s_max)
        probs = p / jnp.sum(p, axis=-1, keepdims=True)
        idxg = idxg_ref[:, :]
        eids = lax.broadcasted_iota(jnp.int32, (1, N_EXP), 1)
        gate = jnp.sum(jnp.where(idxg == eids, probs, 0.0), axis=-1,
                       keepdims=True)

        parts = []
        for el in range(E_LOC):
            c = jnp.where(idxg == me * E_LOC + el, gate, 0.0)
            parts.append((xg_ref[:, :] * c).astype(jnp.bfloat16))
        xcat = jnp.concatenate(parts, axis=1)
        wcat = jnp.reshape(ew_ref[:, :, :],
                           (E_LOC * D_MODEL, D_FF)).astype(jnp.bfloat16)
        partial_ref[:, :, :] = jnp.reshape(
            jnp.dot(xcat, wcat, preferred_element_type=jnp.float32
                    ).astype(jnp.bfloat16), (MAX_M, 1, D_FF))

        def issue(i, carry):
            t = tok_sm[i]
            dst = t // BLK
            off = lax.rem(t, BLK)

            @pl.when(dst != me)
            def _():
                rdma = pltpu.make_async_remote_copy(
                    src_ref=partial_ref.at[pl.ds(i, 1)],
                    dst_ref=comm_ref.at[pl.ds(off, 1)],
                    send_sem=ssem,
                    recv_sem=rsem,
                    device_id=(dst,),
                    device_id_type=pl.DeviceIdType.MESH,
                )
                rdma.start()

            @pl.when(dst == me)
            def _():
                comm_ref[pl.ds(off, 1)] = partial_ref[pl.ds(i, 1)]

            return carry

        lax.fori_loop(0, m_cnt, issue, 0)

        shared = jnp.dot(xm_ref[:, :].astype(jnp.bfloat16),
                         sw_ref[:, :].astype(jnp.bfloat16),
                         preferred_element_type=jnp.float32)

        def wait_one_recv(i, carry):
            recv = pltpu.make_async_remote_copy(
                src_ref=partial_ref.at[pl.ds(0, 1)],
                dst_ref=comm_ref.at[pl.ds(0, 1)],
                send_sem=ssem,
                recv_sem=rsem,
                device_id=(me,),
                device_id_type=pl.DeviceIdType.MESH,
            )
            recv.wait_recv()
            return carry

        lax.fori_loop(0, n_rcv, wait_one_recv, 0)

        out_ref[:, :] = shared + jnp.reshape(
            comm_ref[:, :, :], (BLK, D_FF)).astype(jnp.float32)

        def wait_one_send(i, carry):
            snd = pltpu.make_async_remote_copy(
                src_ref=partial_ref.at[pl.ds(0, 1)],
                dst_ref=comm_ref.at[pl.ds(0, 1)],
                send_sem=ssem,
                recv_sem=rsem,
                device_id=(me,),
                device_id_type=pl.DeviceIdType.MESH,
            )
            snd.wait_send()
            return carry

        lax.fori_loop(0, n_snd, wait_one_send, 0)

    grid_spec = pltpu.PrefetchScalarGridSpec(
        num_scalar_prefetch=2,
        in_specs=[pl.BlockSpec(memory_space=pltpu.VMEM)] * 6,
        out_specs=pl.BlockSpec(memory_space=pltpu.VMEM),
        scratch_shapes=[
            pltpu.VMEM((MAX_M, 1, D_FF), jnp.bfloat16),
            pltpu.VMEM((BLK, 1, D_FF), jnp.bfloat16),
            pltpu.SemaphoreType.DMA,
            pltpu.SemaphoreType.DMA,
        ],
    )
    return pl.pallas_call(
        body,
        out_shape=jax.ShapeDtypeStruct((BLK, D_FF), jnp.float32),
        grid_spec=grid_spec,
        compiler_params=pltpu.CompilerParams(collective_id=0),
    )(tok, meta, xg, idxg, x_mine, router_W, expert_W, shared_W)
